# Initial kernel scaffold; baseline (speedup 1.0000x reference)
import functools

import jax
import jax.numpy as jnp
from jax import lax
from jax.experimental import pallas as pl
from jax.experimental.pallas import tpu as pltpu
from jax.experimental.pallas import tpu_sc as plsc

N = 10000
E = 320000
NC = 2
NS = 16
NW = NC * NS
E_PER_CORE = E // NC
E_PER_TILE = E // NW
K = 80
NCHUNK = E_PER_TILE // K
ROWS_PER_TILE = N // NS
DEG_W = 16


def _sc_agg(z, src, dst, zero_rows, zero_deg, with_deg):
    D = z.shape[1]
    mesh = plsc.VectorSubcoreMesh(
        core_axis_name="c", subcore_axis_name="s", num_cores=NC,
        num_subcores=NS)

    out_type = [jax.ShapeDtypeStruct((NC, N, D), jnp.float32)]
    scratch = [
        pltpu.VMEM((K,), jnp.int32),
        pltpu.VMEM((K,), jnp.int32),
        pltpu.VMEM((K, D), jnp.float32),
        pltpu.VMEM_SHARED((N, D), jnp.float32),
        pltpu.SemaphoreType.DMA,
    ]
    if with_deg:
        out_type.append(jax.ShapeDtypeStruct((NC, N, DEG_W), jnp.float32))
        scratch += [
            pltpu.VMEM((K, DEG_W), jnp.float32),
            pltpu.VMEM_SHARED((N, DEG_W), jnp.float32),
        ]

    @functools.partial(pl.kernel, out_type=out_type, mesh=mesh,
                       scratch_types=scratch)
    def agg(z_hbm, src_hbm, dst_hbm, zrow_hbm, zdeg_hbm, *rest):
        if with_deg:
            (acc_out, deg_out, src_v, dst_v, rows_v, acc_sh, sem,
             ones_v, deg_sh) = rest
        else:
            acc_out, src_v, dst_v, rows_v, acc_sh, sem = rest
        c = lax.axis_index("c")
        s = lax.axis_index("s")
        base = c * E_PER_CORE + s * E_PER_TILE

        r0 = s * ROWS_PER_TILE
        pltpu.sync_copy(zrow_hbm, acc_sh.at[pl.ds(r0, ROWS_PER_TILE)])
        if with_deg:
            pltpu.sync_copy(zdeg_hbm, deg_sh.at[pl.ds(r0, ROWS_PER_TILE)])
            ones = jnp.full((DEG_W,), 1.0, jnp.float32)
            for i in range(K):
                ones_v[i] = ones
        plsc.subcore_barrier()

        def body(j, carry):
            off = base + j * K
            pltpu.sync_copy(src_hbm.at[pl.ds(off, K)], src_v)
            pltpu.sync_copy(dst_hbm.at[pl.ds(off, K)], dst_v)
            pltpu.async_copy(z_hbm.at[src_v], rows_v, sem).wait()
            pltpu.sync_copy(rows_v, acc_sh.at[dst_v], add=True)
            if with_deg:
                pltpu.sync_copy(ones_v, deg_sh.at[dst_v], add=True)
            return carry

        lax.fori_loop(0, NCHUNK, body, 0)
        plsc.subcore_barrier()

        pltpu.sync_copy(acc_sh.at[pl.ds(r0, ROWS_PER_TILE)],
                        acc_out.at[c, pl.ds(r0, ROWS_PER_TILE)])
        if with_deg:
            pltpu.sync_copy(deg_sh.at[pl.ds(r0, ROWS_PER_TILE)],
                            deg_out.at[c, pl.ds(r0, ROWS_PER_TILE)])

    return agg(z, src, dst, zero_rows, zero_deg)


ROW_BLK = 1000


def _tc_stage0(h, W_self0, W_neigh0, b0):
    def body(h_ref, ws_ref, wn_ref, b_ref, s_ref, z_ref):
        hblk = h_ref[...]
        s_ref[...] = jnp.dot(hblk, ws_ref[...],
                             preferred_element_type=jnp.float32) + b_ref[...]
        z_ref[...] = jnp.dot(hblk, wn_ref[...],
                             preferred_element_type=jnp.float32)

    H = W_self0.shape[1]
    return pl.pallas_call(
        body,
        grid=(N // ROW_BLK,),
        in_specs=[
            pl.BlockSpec((ROW_BLK, h.shape[1]), lambda i: (i, 0)),
            pl.BlockSpec(W_self0.shape, lambda i: (0, 0)),
            pl.BlockSpec(W_neigh0.shape, lambda i: (0, 0)),
            pl.BlockSpec((1, H), lambda i: (0, 0)),
        ],
        out_specs=[
            pl.BlockSpec((ROW_BLK, H), lambda i: (i, 0)),
            pl.BlockSpec((ROW_BLK, H), lambda i: (i, 0)),
        ],
        out_shape=[
            jax.ShapeDtypeStruct((N, H), jnp.float32),
            jax.ShapeDtypeStruct((N, H), jnp.float32),
        ],
    )(h, W_self0, W_neigh0, b0)


def _tc_stage1(s0, acc0, deg, W_self1, W_neigh1, b1):
    def body(s0_ref, acc_ref, deg_ref, ws_ref, wn_ref, b_ref, s_ref, z_ref):
        agg = acc_ref[0] + acc_ref[1]
        d = deg_ref[0, :, 0:1] + deg_ref[1, :, 0:1]
        rdeg = 1.0 / jnp.maximum(d, 1.0)
        h1 = jnp.maximum(s0_ref[...] + agg * rdeg, 0.0)
        s_ref[...] = jnp.dot(h1, ws_ref[...],
                             preferred_element_type=jnp.float32) + b_ref[...]
        z_ref[...] = jnp.dot(h1, wn_ref[...],
                             preferred_element_type=jnp.float32)

    H = s0.shape[1]
    C = W_self1.shape[1]
    return pl.pallas_call(
        body,
        grid=(N // ROW_BLK,),
        in_specs=[
            pl.BlockSpec((ROW_BLK, H), lambda i: (i, 0)),
            pl.BlockSpec((NC, ROW_BLK, H), lambda i: (0, i, 0)),
            pl.BlockSpec((NC, ROW_BLK, DEG_W), lambda i: (0, i, 0)),
            pl.BlockSpec(W_self1.shape, lambda i: (0, 0)),
            pl.BlockSpec(W_neigh1.shape, lambda i: (0, 0)),
            pl.BlockSpec((1, C), lambda i: (0, 0)),
        ],
        out_specs=[
            pl.BlockSpec((ROW_BLK, C), lambda i: (i, 0)),
            pl.BlockSpec((ROW_BLK, C), lambda i: (i, 0)),
        ],
        out_shape=[
            jax.ShapeDtypeStruct((N, C), jnp.float32),
            jax.ShapeDtypeStruct((N, C), jnp.float32),
        ],
    )(s0, acc0, deg, W_self1, W_neigh1, b1)


def _tc_stage2(s1, acc1, deg):
    def body(s1_ref, acc_ref, deg_ref, o_ref):
        agg = acc_ref[0] + acc_ref[1]
        d = deg_ref[0, :, 0:1] + deg_ref[1, :, 0:1]
        rdeg = 1.0 / jnp.maximum(d, 1.0)
        o_ref[...] = s1_ref[...] + agg * rdeg

    C = s1.shape[1]
    return pl.pallas_call(
        body,
        grid=(N // ROW_BLK,),
        in_specs=[
            pl.BlockSpec((ROW_BLK, C), lambda i: (i, 0)),
            pl.BlockSpec((NC, ROW_BLK, C), lambda i: (0, i, 0)),
            pl.BlockSpec((NC, ROW_BLK, DEG_W), lambda i: (0, i, 0)),
        ],
        out_specs=pl.BlockSpec((ROW_BLK, C), lambda i: (i, 0)),
        out_shape=jax.ShapeDtypeStruct((N, C), jnp.float32),
    )(s1, acc1, deg)


def kernel(h, edge_index, W_self0, W_neigh0, b0, W_self1, W_neigh1, b1):
    src = edge_index[0]
    dst = edge_index[1]
    zero128 = jnp.zeros((ROWS_PER_TILE, 128), jnp.float32)
    zero64 = jnp.zeros((ROWS_PER_TILE, 64), jnp.float32)
    zero_deg = jnp.zeros((ROWS_PER_TILE, DEG_W), jnp.float32)

    s0, z0 = _tc_stage0(h, W_self0, W_neigh0, b0.reshape(1, -1))
    acc0, deg = _sc_agg(z0, src, dst, zero128, zero_deg, with_deg=True)
    s1, z1 = _tc_stage1(s0, acc0, deg, W_self1, W_neigh1, b1.reshape(1, -1))
    acc1 = _sc_agg(z1, src, dst, zero64, zero_deg, with_deg=False)
    return _tc_stage2(s1, acc1, deg)


# speedup vs baseline: 5.4396x; 5.4396x over previous
import functools

import jax
import jax.numpy as jnp
from jax import lax
from jax.experimental import pallas as pl
from jax.experimental.pallas import tpu as pltpu
from jax.experimental.pallas import tpu_sc as plsc

N = 10000
E = 320000
NC = 2
NS = 16
NW = NC * NS
E_PER_CORE = E // NC
E_PER_TILE = E // NW
K = 80
NCHUNK = E_PER_TILE // K
N_PAD = 10240
ROWS_PER_TILE = N_PAD // NS


def _sc_agg(z, src, dst, zero_rows, zero_deg, with_deg):
    D = z.shape[1]
    mesh = plsc.VectorSubcoreMesh(
        core_axis_name="c", subcore_axis_name="s", num_cores=NC,
        num_subcores=NS)

    out_type = [jax.ShapeDtypeStruct((NC, N_PAD, D), jnp.float32)]
    scratch = [
        pltpu.VMEM((K,), jnp.int32),
        pltpu.VMEM((K,), jnp.int32),
        pltpu.VMEM((K, D), jnp.float32),
        pltpu.VMEM_SHARED((N_PAD, D), jnp.float32),
        pltpu.SemaphoreType.DMA,
    ]
    if with_deg:
        out_type.append(jax.ShapeDtypeStruct((NW, N_PAD), jnp.float32))
        scratch.append(pltpu.VMEM((N_PAD,), jnp.float32))

    @functools.partial(
        pl.kernel, out_type=out_type, mesh=mesh, scratch_types=scratch,
        compiler_params=pltpu.CompilerParams(needs_layout_passes=False))
    def agg(z_hbm, src_hbm, dst_hbm, zrow_hbm, zdeg_hbm, *rest):
        if with_deg:
            acc_out, deg_out, src_v, dst_v, rows_v, acc_sh, sem, deg_v = rest
        else:
            acc_out, src_v, dst_v, rows_v, acc_sh, sem = rest
        c = lax.axis_index("c")
        s = lax.axis_index("s")
        base = c * E_PER_CORE + s * E_PER_TILE

        r0 = s * ROWS_PER_TILE
        pltpu.sync_copy(zrow_hbm, rows_v)
        for t in range(ROWS_PER_TILE // K):
            pltpu.sync_copy(rows_v, acc_sh.at[pl.ds(r0 + t * K, K)])
        if with_deg:
            pltpu.sync_copy(zdeg_hbm, deg_v)
            ones16 = jnp.full((16,), 1.0, jnp.float32)
        plsc.subcore_barrier()

        def body(j, carry):
            off = base + j * K
            pltpu.sync_copy(src_hbm.at[pl.ds(off, K)], src_v)
            pltpu.sync_copy(dst_hbm.at[pl.ds(off, K)], dst_v)
            pltpu.async_copy(z_hbm.at[src_v], rows_v, sem).wait()
            pltpu.sync_copy(rows_v, acc_sh.at[dst_v], add=True)
            if with_deg:
                for u in range(K // 16):
                    idx16 = dst_v[pl.ds(u * 16, 16)]
                    plsc.addupdate_scatter(deg_v, [idx16], ones16)
            return carry

        lax.fori_loop(0, NCHUNK, body, 0)
        plsc.subcore_barrier()

        for t in range(ROWS_PER_TILE // K):
            pltpu.sync_copy(acc_sh.at[pl.ds(r0 + t * K, K)], rows_v)
            pltpu.sync_copy(rows_v, acc_out.at[c, pl.ds(r0 + t * K, K)])
        if with_deg:
            pltpu.sync_copy(deg_v, deg_out.at[c * NS + s])

    res = agg(z, src, dst, zero_rows, zero_deg)
    if not isinstance(res, (list, tuple)):
        res = (res,)
    return res[0] if not with_deg else tuple(res)


ROW_BLK = 1024
GRID = N_PAD // ROW_BLK


def _deg_col(deg_blk):
    ones = jnp.ones((NW, 1), jnp.float32)
    d = lax.dot_general(deg_blk, ones, (((0,), (0,)), ((), ())),
                        preferred_element_type=jnp.float32)
    return jnp.maximum(d, 1.0)


def _tc_stage0(h, W_self0, W_neigh0, b0):
    def body(h_ref, ws_ref, wn_ref, b_ref, s_ref, z_ref):
        hblk = h_ref[...]
        s_ref[...] = jnp.dot(hblk, ws_ref[...],
                             preferred_element_type=jnp.float32) + b_ref[...]
        z_ref[...] = jnp.dot(hblk, wn_ref[...],
                             preferred_element_type=jnp.float32)

    H = W_self0.shape[1]
    return pl.pallas_call(
        body,
        grid=(GRID,),
        in_specs=[
            pl.BlockSpec((ROW_BLK, h.shape[1]), lambda i: (i, 0)),
            pl.BlockSpec(W_self0.shape, lambda i: (0, 0)),
            pl.BlockSpec(W_neigh0.shape, lambda i: (0, 0)),
            pl.BlockSpec((1, H), lambda i: (0, 0)),
        ],
        out_specs=[
            pl.BlockSpec((ROW_BLK, H), lambda i: (i, 0)),
            pl.BlockSpec((ROW_BLK, H), lambda i: (i, 0)),
        ],
        out_shape=[
            jax.ShapeDtypeStruct((N_PAD, H), jnp.float32),
            jax.ShapeDtypeStruct((N_PAD, H), jnp.float32),
        ],
    )(h, W_self0, W_neigh0, b0)


def _tc_stage1(s0, acc0, deg, W_self1, W_neigh1, b1):
    def body(s0_ref, acc_ref, deg_ref, ws_ref, wn_ref, b_ref, s_ref, z_ref):
        agg = acc_ref[0] + acc_ref[1]
        rdeg = 1.0 / _deg_col(deg_ref[...])
        h1 = jnp.maximum(s0_ref[...] + agg * rdeg, 0.0)
        s_ref[...] = jnp.dot(h1, ws_ref[...],
                             preferred_element_type=jnp.float32) + b_ref[...]
        z_ref[...] = jnp.dot(h1, wn_ref[...],
                             preferred_element_type=jnp.float32)

    H = s0.shape[1]
    C = W_self1.shape[1]
    return pl.pallas_call(
        body,
        grid=(GRID,),
        in_specs=[
            pl.BlockSpec((ROW_BLK, H), lambda i: (i, 0)),
            pl.BlockSpec((NC, ROW_BLK, H), lambda i: (0, i, 0)),
            pl.BlockSpec((NW, ROW_BLK), lambda i: (0, i)),
            pl.BlockSpec(W_self1.shape, lambda i: (0, 0)),
            pl.BlockSpec(W_neigh1.shape, lambda i: (0, 0)),
            pl.BlockSpec((1, C), lambda i: (0, 0)),
        ],
        out_specs=[
            pl.BlockSpec((ROW_BLK, C), lambda i: (i, 0)),
            pl.BlockSpec((ROW_BLK, C), lambda i: (i, 0)),
        ],
        out_shape=[
            jax.ShapeDtypeStruct((N_PAD, C), jnp.float32),
            jax.ShapeDtypeStruct((N_PAD, C), jnp.float32),
        ],
    )(s0, acc0, deg, W_self1, W_neigh1, b1)


def _tc_stage2(s1, acc1, deg):
    def body(s1_ref, acc_ref, deg_ref, o_ref):
        agg = acc_ref[0] + acc_ref[1]
        rdeg = 1.0 / _deg_col(deg_ref[...])
        o_ref[...] = s1_ref[...] + agg * rdeg

    C = s1.shape[1]
    return pl.pallas_call(
        body,
        grid=(GRID,),
        in_specs=[
            pl.BlockSpec((ROW_BLK, C), lambda i: (i, 0)),
            pl.BlockSpec((NC, ROW_BLK, C), lambda i: (0, i, 0)),
            pl.BlockSpec((NW, ROW_BLK), lambda i: (0, i)),
        ],
        out_specs=pl.BlockSpec((ROW_BLK, C), lambda i: (i, 0)),
        out_shape=jax.ShapeDtypeStruct((N_PAD, C), jnp.float32),
    )(s1, acc1, deg)


def kernel(h, edge_index, W_self0, W_neigh0, b0, W_self1, W_neigh1, b1):
    src = edge_index[0]
    dst = edge_index[1]
    zero128 = jnp.zeros((K, 128), jnp.float32)
    zero_deg = jnp.zeros((N_PAD,), jnp.float32)

    C = W_self1.shape[1]
    pad = ((0, 0), (0, 128 - C))
    Ws1 = jnp.pad(W_self1, pad)
    Wn1 = jnp.pad(W_neigh1, pad)
    b1p = jnp.pad(b1, ((0, 128 - C),))

    s0, z0 = _tc_stage0(h, W_self0, W_neigh0, b0.reshape(1, -1))
    acc0, deg = _sc_agg(z0, src, dst, zero128, zero_deg, with_deg=True)
    s1, z1 = _tc_stage1(s0, acc0, deg, Ws1, Wn1, b1p.reshape(1, -1))
    acc1 = _sc_agg(z1, src, dst, zero128, zero_deg, with_deg=False)
    return _tc_stage2(s1, acc1, deg)[:N, :C]


# speedup vs baseline: 7.6266x; 1.4020x over previous
import functools

import jax
import jax.numpy as jnp
from jax import lax
from jax.experimental import pallas as pl
from jax.experimental.pallas import tpu as pltpu
from jax.experimental.pallas import tpu_sc as plsc

N = 10000
E = 320000
NC = 2
NS = 16
NW = NC * NS
E_PER_CORE = E // NC
E_PER_TILE = E // NW
K = 80
NCHUNK = E_PER_TILE // K
N_PAD = 10240
ROWS_PER_TILE = N_PAD // NS


def _sc_agg(z, src, dst, zero_rows, zero_deg, with_deg):
    D = z.shape[1]
    mesh = plsc.VectorSubcoreMesh(
        core_axis_name="c", subcore_axis_name="s", num_cores=NC,
        num_subcores=NS)

    out_type = [jax.ShapeDtypeStruct((NC, N_PAD, D), jnp.float32)]
    scratch = [
        pltpu.VMEM((E_PER_TILE,), jnp.int32),
        pltpu.VMEM((NCHUNK, K), jnp.int32),
        pltpu.VMEM((K, D), jnp.float32),
        pltpu.VMEM_SHARED((N_PAD, D), jnp.float32),
        pltpu.SemaphoreType.DMA,
    ]
    if with_deg:
        out_type.append(jax.ShapeDtypeStruct((NW, N_PAD), jnp.float32))
        scratch.append(pltpu.VMEM((N_PAD,), jnp.float32))

    @functools.partial(
        pl.kernel, out_type=out_type, mesh=mesh, scratch_types=scratch,
        compiler_params=pltpu.CompilerParams(needs_layout_passes=False))
    def agg(z_hbm, src_hbm, dst3_hbm, zrow_hbm, zdeg_hbm, *rest):
        if with_deg:
            acc_out, deg_out, src_v, dst_v, rows_v, acc_sh, sem, deg_v = rest
        else:
            acc_out, src_v, dst_v, rows_v, acc_sh, sem = rest
        c = lax.axis_index("c")
        s = lax.axis_index("s")
        w = c * NS + s
        base = c * E_PER_CORE + s * E_PER_TILE

        pltpu.sync_copy(src_hbm.at[pl.ds(base, E_PER_TILE)], src_v)
        pltpu.sync_copy(dst3_hbm.at[w], dst_v)

        r0 = s * ROWS_PER_TILE
        pltpu.sync_copy(zrow_hbm, rows_v)
        for t in range(ROWS_PER_TILE // K):
            pltpu.sync_copy(rows_v, acc_sh.at[pl.ds(r0 + t * K, K)])
        if with_deg:
            pltpu.sync_copy(zdeg_hbm, deg_v)
            ones16 = jnp.full((16,), 1.0, jnp.float32)
        plsc.subcore_barrier()

        def body(j, carry):
            pltpu.async_copy(
                z_hbm.at[src_v.at[pl.ds(j * K, K)]], rows_v, sem).wait()
            pltpu.sync_copy(rows_v, acc_sh.at[dst_v.at[j]], add=True)
            if with_deg:
                for u in range(K // 16):
                    idx16 = dst_v[j, pl.ds(u * 16, 16)]
                    plsc.addupdate_scatter(deg_v, [idx16], ones16)
            return carry

        lax.fori_loop(0, NCHUNK, body, 0)
        plsc.subcore_barrier()

        for t in range(ROWS_PER_TILE // K):
            pltpu.sync_copy(acc_sh.at[pl.ds(r0 + t * K, K)], rows_v)
            pltpu.sync_copy(rows_v, acc_out.at[c, pl.ds(r0 + t * K, K)])
        if with_deg:
            pltpu.sync_copy(deg_v, deg_out.at[w])

    res = agg(z, src, dst, zero_rows, zero_deg)
    if not isinstance(res, (list, tuple)):
        res = (res,)
    return res[0] if not with_deg else tuple(res)


ROW_BLK = 1024
GRID = N_PAD // ROW_BLK


def _deg_col(deg_blk):
    ones = jnp.ones((NW, 1), jnp.float32)
    d = lax.dot_general(deg_blk, ones, (((0,), (0,)), ((), ())),
                        preferred_element_type=jnp.float32)
    return jnp.maximum(d, 1.0)


def _tc_stage0(h, W_self0, W_neigh0, b0):
    def body(h_ref, ws_ref, wn_ref, b_ref, s_ref, z_ref):
        hblk = h_ref[...]
        s_ref[...] = jnp.dot(hblk, ws_ref[...],
                             preferred_element_type=jnp.float32) + b_ref[...]
        z_ref[...] = jnp.dot(hblk, wn_ref[...],
                             preferred_element_type=jnp.float32)

    H = W_self0.shape[1]
    return pl.pallas_call(
        body,
        grid=(GRID,),
        in_specs=[
            pl.BlockSpec((ROW_BLK, h.shape[1]), lambda i: (i, 0)),
            pl.BlockSpec(W_self0.shape, lambda i: (0, 0)),
            pl.BlockSpec(W_neigh0.shape, lambda i: (0, 0)),
            pl.BlockSpec((1, H), lambda i: (0, 0)),
        ],
        out_specs=[
            pl.BlockSpec((ROW_BLK, H), lambda i: (i, 0)),
            pl.BlockSpec((ROW_BLK, H), lambda i: (i, 0)),
        ],
        out_shape=[
            jax.ShapeDtypeStruct((N_PAD, H), jnp.float32),
            jax.ShapeDtypeStruct((N_PAD, H), jnp.float32),
        ],
    )(h, W_self0, W_neigh0, b0)


def _tc_stage1(s0, acc0, deg, W_self1, W_neigh1, b1):
    def body(s0_ref, acc_ref, deg_ref, ws_ref, wn_ref, b_ref, s_ref, z_ref):
        agg = acc_ref[0] + acc_ref[1]
        rdeg = 1.0 / _deg_col(deg_ref[...])
        h1 = jnp.maximum(s0_ref[...] + agg * rdeg, 0.0)
        s_ref[...] = jnp.dot(h1, ws_ref[...],
                             preferred_element_type=jnp.float32) + b_ref[...]
        z_ref[...] = jnp.dot(h1, wn_ref[...],
                             preferred_element_type=jnp.float32)

    H = s0.shape[1]
    C = W_self1.shape[1]
    return pl.pallas_call(
        body,
        grid=(GRID,),
        in_specs=[
            pl.BlockSpec((ROW_BLK, H), lambda i: (i, 0)),
            pl.BlockSpec((NC, ROW_BLK, H), lambda i: (0, i, 0)),
            pl.BlockSpec((NW, ROW_BLK), lambda i: (0, i)),
            pl.BlockSpec(W_self1.shape, lambda i: (0, 0)),
            pl.BlockSpec(W_neigh1.shape, lambda i: (0, 0)),
            pl.BlockSpec((1, C), lambda i: (0, 0)),
        ],
        out_specs=[
            pl.BlockSpec((ROW_BLK, C), lambda i: (i, 0)),
            pl.BlockSpec((ROW_BLK, C), lambda i: (i, 0)),
        ],
        out_shape=[
            jax.ShapeDtypeStruct((N_PAD, C), jnp.float32),
            jax.ShapeDtypeStruct((N_PAD, C), jnp.float32),
        ],
    )(s0, acc0, deg, W_self1, W_neigh1, b1)


def _tc_stage2(s1, acc1, deg):
    def body(s1_ref, acc_ref, deg_ref, o_ref):
        agg = acc_ref[0] + acc_ref[1]
        rdeg = 1.0 / _deg_col(deg_ref[...])
        o_ref[...] = s1_ref[...] + agg * rdeg

    C = s1.shape[1]
    return pl.pallas_call(
        body,
        grid=(GRID,),
        in_specs=[
            pl.BlockSpec((ROW_BLK, C), lambda i: (i, 0)),
            pl.BlockSpec((NC, ROW_BLK, C), lambda i: (0, i, 0)),
            pl.BlockSpec((NW, ROW_BLK), lambda i: (0, i)),
        ],
        out_specs=pl.BlockSpec((ROW_BLK, C), lambda i: (i, 0)),
        out_shape=jax.ShapeDtypeStruct((N_PAD, C), jnp.float32),
    )(s1, acc1, deg)


def kernel(h, edge_index, W_self0, W_neigh0, b0, W_self1, W_neigh1, b1):
    src = edge_index[0]
    dst = edge_index[1].reshape(NW, NCHUNK, K)
    zero128 = jnp.zeros((K, 128), jnp.float32)
    zero_deg = jnp.zeros((N_PAD,), jnp.float32)

    C = W_self1.shape[1]
    pad = ((0, 0), (0, 128 - C))
    Ws1 = jnp.pad(W_self1, pad)
    Wn1 = jnp.pad(W_neigh1, pad)
    b1p = jnp.pad(b1, ((0, 128 - C),))

    s0, z0 = _tc_stage0(h, W_self0, W_neigh0, b0.reshape(1, -1))
    acc0, deg = _sc_agg(z0, src, dst, zero128, zero_deg, with_deg=True)
    s1, z1 = _tc_stage1(s0, acc0, deg, Ws1, Wn1, b1p.reshape(1, -1))
    acc1 = _sc_agg(z1, src, dst, zero128, zero_deg, with_deg=False)
    return _tc_stage2(s1, acc1, deg)[:N, :C]
